# Initial kernel scaffold; baseline (speedup 1.0000x reference)
import jax
import jax.numpy as jnp
from jax.experimental import pallas as pl


def kernel(x, vector_table, position_table):
    raise NotImplementedError("write your pallas kernel here")


# speedup vs baseline: 2.3952x; 2.3952x over previous
import functools

import jax
import jax.numpy as jnp
from jax import lax
from jax.experimental import pallas as pl
from jax.experimental.pallas import tpu as pltpu
from jax.experimental.pallas import tpu_sc as plsc

_VOCAB = 100000
_EMBED = 300
_EMBED_PAD = 384
_LENGTH = 200
_BATCH = 4096
_TOTAL = _LENGTH * _BATCH

_NC = 2
_NS = 16
_NW = _NC * _NS
_PER_W = _TOTAL // _NW
_CHUNK = 128
_NCHUNK = _PER_W // _CHUNK

_ADD_ROWS = 1000


def _add_body(v_ref, p_ref, o_ref):
    o_ref[:, : _EMBED] = v_ref[...] + p_ref[...]
    o_ref[:, _EMBED :] = jnp.zeros(
        (_ADD_ROWS, _EMBED_PAD - _EMBED), jnp.float32
    )


def _combine_tables(vector_table, position_table):
    in_spec = pl.BlockSpec((_ADD_ROWS, _EMBED), lambda i: (i, 0))
    out_spec = pl.BlockSpec((_ADD_ROWS, _EMBED_PAD), lambda i: (i, 0))
    return pl.pallas_call(
        _add_body,
        grid=(_VOCAB // _ADD_ROWS,),
        in_specs=[in_spec, in_spec],
        out_specs=out_spec,
        out_shape=jax.ShapeDtypeStruct((_VOCAB, _EMBED_PAD), jnp.float32),
    )(vector_table, position_table)


def _gather_body(tbl_hbm, idx_hbm, out_hbm, idx_v, buf_v, gsem):
    wid = lax.axis_index("s") * _NC + lax.axis_index("c")
    base = wid * _PER_W
    pltpu.sync_copy(idx_hbm.at[wid], idx_v)

    def body(j, carry):
        pltpu.async_copy(tbl_hbm.at[idx_v.at[j]], buf_v, gsem).wait()
        pltpu.sync_copy(buf_v, out_hbm.at[pl.ds(base + j * _CHUNK, _CHUNK)])
        return carry

    lax.fori_loop(0, _NCHUNK, body, 0)


def _make_gather():
    return functools.partial(
        pl.kernel,
        out_type=jax.ShapeDtypeStruct((_TOTAL, _EMBED_PAD), jnp.float32),
        mesh=plsc.VectorSubcoreMesh(core_axis_name="c", subcore_axis_name="s"),
        scratch_types=[
            pltpu.VMEM((_NCHUNK, _CHUNK), jnp.int32),
            pltpu.VMEM((_CHUNK, _EMBED_PAD), jnp.float32),
            pltpu.SemaphoreType.DMA,
        ],
    )(_gather_body)


def kernel(x, vector_table, position_table):
    sum_table = _combine_tables(vector_table, position_table)
    xf = x.reshape(-1).astype(jnp.int32).reshape(_NW, _NCHUNK, _CHUNK)
    out = _make_gather()(sum_table, xf)
    return out[:, : _EMBED].reshape(_LENGTH, _BATCH, _EMBED)
